# baseline (speedup 1.0000x reference)
import functools

import jax
import jax.numpy as jnp
from jax import lax
from jax.experimental import pallas as pl
from jax.experimental.pallas import tpu as pltpu
from jax.experimental.pallas import tpu_sc as plsc

_NUM_CORES = 2
_NUM_SUBCORES = 16
_NUM_WORKERS = _NUM_CORES * _NUM_SUBCORES
_ROWS_PER_CHUNK = 8
_BUILD_BLOCK = 32768
_LANES = 16


def _build_table(weight):
    num_rows, dim = weight.shape
    wt = weight.T

    def body(wt_ref, out_ref):
        out_ref[:, 0:dim] = wt_ref[...].T

    return pl.pallas_call(
        body,
        grid=(pl.cdiv(num_rows, _BUILD_BLOCK),),
        in_specs=[pl.BlockSpec((dim, _BUILD_BLOCK), lambda i: (0, i))],
        out_specs=pl.BlockSpec((_BUILD_BLOCK, 128), lambda i: (i, 0)),
        out_shape=jax.ShapeDtypeStruct((num_rows, 128), weight.dtype),
    )(wt)


_SPLITS = 2


def _gather_rows(wpad, flat_ids, batch, seq, dim):
    pad_dim = wpad.shape[1]
    rows_per_worker = batch // _NUM_WORKERS
    chunk = _ROWS_PER_CHUNK * seq
    nchunks = rows_per_worker // _ROWS_PER_CHUNK
    mesh = plsc.VectorSubcoreMesh(core_axis_name="c", subcore_axis_name="s")

    @functools.partial(
        pl.kernel,
        mesh=mesh,
        out_type=jax.ShapeDtypeStruct((batch, seq, dim), wpad.dtype),
        scratch_types=[
            pltpu.VMEM((chunk,), jnp.int32),
            pltpu.VMEM((chunk,), jnp.int32),
            pltpu.VMEM((chunk, pad_dim), wpad.dtype),
            pltpu.VMEM((chunk, pad_dim), wpad.dtype),
            pltpu.VMEM((_ROWS_PER_CHUNK, seq, dim), wpad.dtype),
            pltpu.SemaphoreType.DMA,
            pltpu.SemaphoreType.DMA,
        ],
    )
    def gather_kernel(
        table_hbm, idx_hbm, out_hbm,
        idx0, idx1, rows0, rows1, rows64, sem0, sem1,
    ):
        wid = lax.axis_index("s") * _NUM_CORES + lax.axis_index("c")
        row0 = wid * rows_per_worker

        def start(k, idx_v, rows_v, sem):
            pltpu.sync_copy(idx_hbm.at[pl.ds((row0 + k * _ROWS_PER_CHUNK) * seq, chunk)], idx_v)
            pltpu.async_copy(table_hbm.at[idx_v], rows_v, sem)

        def finish(k, idx_v, rows_v, sem):
            pltpu.make_async_copy(table_hbm.at[idx_v], rows_v, sem).wait()

            @pl.loop(0, _ROWS_PER_CHUNK)
            def _(j):
                @pl.loop(0, seq)
                def _(s):
                    for c in range(dim // _LANES):
                        rows64.at[j, s][pl.ds(c * _LANES, _LANES)] = (
                            rows_v.at[j * seq + s][pl.ds(c * _LANES, _LANES)]
                        )

            pltpu.sync_copy(
                rows64,
                out_hbm.at[pl.ds(row0 + k * _ROWS_PER_CHUNK, _ROWS_PER_CHUNK)],
            )

        start(0, idx0, rows0, sem0)

        @pl.loop(0, nchunks, step=2)
        def _(k):
            @pl.when(k + 1 < nchunks)
            def _():
                start(k + 1, idx1, rows1, sem1)

            finish(k, idx0, rows0, sem0)

            @pl.when(k + 2 < nchunks)
            def _():
                start(k + 2, idx0, rows0, sem0)

            @pl.when(k + 1 < nchunks)
            def _():
                finish(k + 1, idx1, rows1, sem1)

    return gather_kernel(wpad, flat_ids)


def kernel(token_ids, weight):
    batch, seq = token_ids.shape
    dim = weight.shape[1]
    wpad = _build_table(weight)
    half = seq // _SPLITS
    parts = [
        _gather_rows(
            wpad,
            token_ids[:, h * half:(h + 1) * half].reshape(batch * half),
            batch, half, dim,
        )
        for h in range(_SPLITS)
    ]
    return jnp.concatenate(parts, axis=1)


# speedup vs baseline: 1.3216x; 1.3216x over previous
import functools

import jax
import jax.numpy as jnp
from jax import lax
from jax.experimental import pallas as pl
from jax.experimental.pallas import tpu as pltpu
from jax.experimental.pallas import tpu_sc as plsc

_NUM_CORES = 2
_NUM_SUBCORES = 16
_NUM_WORKERS = _NUM_CORES * _NUM_SUBCORES
_ROWS_PER_CHUNK = 16
_BUILD_BLOCK = 32768


def _build_table(weight):
    num_rows, dim = weight.shape
    wt = weight.T

    def body(wt_ref, out_ref):
        out_ref[:, 0:dim] = wt_ref[...].T

    return pl.pallas_call(
        body,
        grid=(pl.cdiv(num_rows, _BUILD_BLOCK),),
        in_specs=[pl.BlockSpec((dim, _BUILD_BLOCK), lambda i: (0, i))],
        out_specs=pl.BlockSpec((_BUILD_BLOCK, 128), lambda i: (i, 0)),
        out_shape=jax.ShapeDtypeStruct((num_rows, 128), weight.dtype),
    )(wt)


def _gather_rows(wpad, flat_ids, batch, seq):
    pad_dim = wpad.shape[1]
    rows_per_worker = batch // _NUM_WORKERS
    chunk = _ROWS_PER_CHUNK * seq
    mesh = plsc.VectorSubcoreMesh(core_axis_name="c", subcore_axis_name="s")

    @functools.partial(
        pl.kernel,
        mesh=mesh,
        out_type=jax.ShapeDtypeStruct((batch, seq, pad_dim), wpad.dtype),
        scratch_types=[
            pltpu.VMEM((chunk,), jnp.int32),
            pltpu.VMEM((chunk, pad_dim), wpad.dtype),
            pltpu.SemaphoreType.DMA,
        ],
    )
    def gather_kernel(table_hbm, idx_hbm, out_hbm, idx_v, rows_v, sem):
        wid = lax.axis_index("s") * _NUM_CORES + lax.axis_index("c")
        row0 = wid * rows_per_worker

        @pl.loop(0, rows_per_worker, step=_ROWS_PER_CHUNK)
        def _(r):
            pltpu.sync_copy(idx_hbm.at[pl.ds((row0 + r) * seq, chunk)], idx_v)
            pltpu.async_copy(table_hbm.at[idx_v], rows_v, sem).wait()
            for j in range(_ROWS_PER_CHUNK):
                pltpu.sync_copy(
                    rows_v.at[pl.ds(j * seq, seq)],
                    out_hbm.at[row0 + r + j],
                )

    return gather_kernel(wpad, flat_ids)


def kernel(token_ids, weight):
    batch, seq = token_ids.shape
    dim = weight.shape[1]
    flat_ids = token_ids.reshape(batch * seq)
    wpad = _build_table(weight)
    out_pad = _gather_rows(wpad, flat_ids, batch, seq)
    return out_pad[:, :, :dim]
